# baseline (speedup 1.0000x reference)
import functools

import jax
import jax.numpy as jnp
from jax import lax
from jax.experimental import pallas as pl
from jax.experimental.pallas import tpu as pltpu
from jax.experimental.pallas import tpu_sc as plsc

_NUM_LEAD_TIMES = 722
_EMBED_DIM = 32
_LANE = 128
_BATCH = 4096

_INFO = plsc.get_sparse_core_info()
_NS = _INFO.num_subcores
_NL = _INFO.num_lanes
_NW = _NS
_B_PER_W = _BATCH // _NW
_CHUNK = 128
_NCHUNK = _B_PER_W // _CHUNK


@functools.partial(
    pl.kernel,
    mesh=plsc.VectorSubcoreMesh(core_axis_name="c", subcore_axis_name="s",
                                num_cores=1),
    out_type=jax.ShapeDtypeStruct((_EMBED_DIM, _BATCH), jnp.float32),
    scratch_types=[
        pltpu.VMEM((_B_PER_W,), jnp.int32),
        pltpu.VMEM((_CHUNK, _LANE), jnp.float32),
        pltpu.VMEM((_CHUNK, _LANE), jnp.float32),
        pltpu.VMEM((_EMBED_DIM, _B_PER_W), jnp.float32),
        pltpu.SemaphoreType.DMA,
        pltpu.SemaphoreType.DMA,
    ],
    compiler_params=pltpu.CompilerParams(needs_layout_passes=False),
)
def _sc_gather(table_hbm, idx_hbm, out_hbm, idx_v, rows0_v, rows1_v, blk_v,
               sem0, sem1):
    wid = lax.axis_index("s")
    base = wid * _B_PER_W
    pltpu.sync_copy(idx_hbm.at[pl.ds(base, _B_PER_W)], idx_v)
    g0 = pltpu.async_copy(table_hbm.at[idx_v.at[pl.ds(0, _CHUNK)]],
                          rows0_v, sem0)
    g1 = pltpu.async_copy(table_hbm.at[idx_v.at[pl.ds(_CHUNK, _CHUNK)]],
                          rows1_v, sem1)

    def transpose_chunk(rows_v, col_base):
        def body(g, _):
            k16 = lax.iota(jnp.int32, _NL) + g * _NL
            for d in range(_EMBED_DIM):
                col = plsc.load_gather(
                    rows_v, [k16, jnp.full((_NL,), d, jnp.int32)])
                blk_v[d, pl.ds(col_base + g * _NL, _NL)] = col
            return 0
        lax.fori_loop(0, _CHUNK // _NL, body, 0, unroll=False)

    g0.wait()
    transpose_chunk(rows0_v, 0)
    g1.wait()
    transpose_chunk(rows1_v, _CHUNK)
    pltpu.sync_copy(blk_v, out_hbm.at[:, pl.ds(base, _B_PER_W)])


def kernel(lead_times, sparse_inputs, dense_inputs_2496, dense_inputs_4996,
           lead_time_embedding):
    del sparse_inputs, dense_inputs_2496, dense_inputs_4996
    table_wide = jnp.pad(lead_time_embedding,
                         ((0, 0), (0, _LANE - _EMBED_DIM)))
    out_t = _sc_gather(table_wide, lead_times.astype(jnp.int32))
    return out_t.T


# speedup vs baseline: 1.2489x; 1.2489x over previous
import functools

import jax
import jax.numpy as jnp
from jax import lax
from jax.experimental import pallas as pl
from jax.experimental.pallas import tpu as pltpu
from jax.experimental.pallas import tpu_sc as plsc

_NUM_LEAD_TIMES = 722
_EMBED_DIM = 32
_BATCH = 4096

_INFO = plsc.get_sparse_core_info()
_NS = _INFO.num_subcores
_NW = _NS
_B_PER_W = _BATCH // _NW
_CHUNK = 128


@functools.partial(
    pl.kernel,
    mesh=plsc.VectorSubcoreMesh(core_axis_name="c", subcore_axis_name="s",
                                num_cores=1),
    out_type=jax.ShapeDtypeStruct((_BATCH, _EMBED_DIM), jnp.float32),
    scratch_types=[
        pltpu.VMEM((_B_PER_W,), jnp.int32),
        pltpu.VMEM((_CHUNK, _EMBED_DIM), jnp.float32),
        pltpu.VMEM((_CHUNK, _EMBED_DIM), jnp.float32),
        pltpu.SemaphoreType.DMA,
        pltpu.SemaphoreType.DMA,
    ],
    compiler_params=pltpu.CompilerParams(use_tc_tiling_on_sc=False),
)
def _sc_gather(table_hbm, idx_hbm, out_hbm, idx_v, rows0_v, rows1_v,
               sem0, sem1):
    wid = lax.axis_index("s")
    base = wid * _B_PER_W
    pltpu.sync_copy(idx_hbm.at[pl.ds(base, _B_PER_W)], idx_v)
    g0 = pltpu.async_copy(table_hbm.at[idx_v.at[pl.ds(0, _CHUNK)]],
                          rows0_v, sem0)
    g1 = pltpu.async_copy(table_hbm.at[idx_v.at[pl.ds(_CHUNK, _CHUNK)]],
                          rows1_v, sem1)
    g0.wait()
    pltpu.sync_copy(rows0_v, out_hbm.at[pl.ds(base, _CHUNK)])
    g1.wait()
    pltpu.sync_copy(rows1_v, out_hbm.at[pl.ds(base + _CHUNK, _CHUNK)])


def kernel(lead_times, sparse_inputs, dense_inputs_2496, dense_inputs_4996,
           lead_time_embedding):
    del sparse_inputs, dense_inputs_2496, dense_inputs_4996
    return _sc_gather(lead_time_embedding, lead_times.astype(jnp.int32))
